# baseline (speedup 1.0000x reference)
import jax
import jax.numpy as jnp
from jax import lax
from jax.experimental import pallas as pl
from jax.experimental.pallas import tpu as pltpu
from jax.experimental.pallas import tpu_sc as plsc

NC = 2
NS = 16
L = 16
NW = NC * NS

BATCH = 16384
D = 32
W = 128
RPW = 4
NV_W = 1000000 * D // W
BPW = BATCH // NW
CHUNK = 128
NCHUNK = BPW // CHUNK
HALF = BPW // 2


def _mf_body(urids_hbm, irids_hbm, uoff_hbm, ioff_hbm, uemb_hbm, iemb_hbm,
             out_hbm, uridx, iridx, uoffv, ioffv, urows, irows, outv, sem):
    wid = lax.axis_index("s") * NC + lax.axis_index("c")

    pltpu.sync_copy(urids_hbm.at[wid], uridx)
    pltpu.sync_copy(irids_hbm.at[wid], iridx)
    pltpu.sync_copy(uoff_hbm.at[wid], uoffv)
    pltpu.sync_copy(ioff_hbm.at[wid], ioffv)

    last_lane = lax.broadcasted_iota(jnp.int32, (L,), 0) == (L - 1)

    for h in range(2):
        copies = []
        for c in range(NCHUNK // 2):
            copies.append(pltpu.async_copy(
                uemb_hbm.at[uridx.at[h * (NCHUNK // 2) + c]],
                urows.at[pl.ds(c * CHUNK, CHUNK), :], sem))
            copies.append(pltpu.async_copy(
                iemb_hbm.at[iridx.at[h * (NCHUNK // 2) + c]],
                irows.at[pl.ds(c * CHUNK, CHUNK), :], sem))
        for cp in copies:
            cp.wait()

        def group_body(g, _):
            uov = uoffv[pl.ds(h * HALF + g * L, L)]
            iov = ioffv[pl.ds(h * HALF + g * L, L)]
            for j in range(L):
                r = g * L + j
                uo = uov[j]
                io = iov[j]
                acc = (urows[r, pl.ds(uo, L)] * irows[r, pl.ds(io, L)]
                       + urows[r, pl.ds(uo + L, L)]
                       * irows[r, pl.ds(io + L, L)])
                total = plsc.cumsum(acc)
                plsc.store_scatter(
                    outv, [jnp.full((L,), h * HALF + r, jnp.int32)], total,
                    mask=last_lane)
            return 0

        lax.fori_loop(0, HALF // L, group_body, 0)

    pltpu.sync_copy(outv, out_hbm.at[pl.ds(wid * BPW, BPW)])


@jax.jit
def kernel(x, user_embedding, item_embedding):
    uids = x[:, 0].astype(jnp.int32)
    iids = x[:, 1].astype(jnp.int32)
    urids = (uids >> 2).reshape(NW, NCHUNK, CHUNK)
    irids = (iids >> 2).reshape(NW, NCHUNK, CHUNK)
    uoff = ((uids & 3) * D).reshape(NW, BPW)
    ioff = ((iids & 3) * D).reshape(NW, BPW)
    uembW = user_embedding.reshape(NV_W, W)
    iembW = item_embedding.reshape(NV_W, W)
    mesh = plsc.VectorSubcoreMesh(core_axis_name="c", subcore_axis_name="s")
    run = pl.kernel(
        _mf_body,
        out_type=jax.ShapeDtypeStruct((BATCH,), jnp.float32),
        mesh=mesh,
        compiler_params=pltpu.CompilerParams(
            needs_layout_passes=False, use_tc_tiling_on_sc=False),
        scratch_types=[
            pltpu.VMEM((NCHUNK, CHUNK), jnp.int32),
            pltpu.VMEM((NCHUNK, CHUNK), jnp.int32),
            pltpu.VMEM((BPW,), jnp.int32),
            pltpu.VMEM((BPW,), jnp.int32),
            pltpu.VMEM((HALF, W), jnp.float32),
            pltpu.VMEM((HALF, W), jnp.float32),
            pltpu.VMEM((BPW,), jnp.float32),
            pltpu.SemaphoreType.DMA,
        ],
    )
    return run(urids, irids, uoff, ioff, uembW, iembW)
